# baseline (speedup 1.0000x reference)
import functools

import jax
import jax.numpy as jnp
from jax import lax
from jax.experimental import pallas as pl
from jax.experimental.pallas import tpu as pltpu
from jax.experimental.pallas import tpu_sc as plsc

_TIMESTEPS = 100


def _tables():
    scale = 1000.0 / _TIMESTEPS
    betas = jnp.linspace(scale * 0.0001, scale * 0.02, _TIMESTEPS)
    alphas_cumprod = jnp.cumprod(1.0 - betas)
    return (jnp.sqrt(alphas_cumprod).astype(jnp.float32),
            jnp.sqrt(1.0 - alphas_cumprod).astype(jnp.float32))


def _sc_gather_body(t_hbm, ta_hbm, tb_hbm, ca_hbm, cb_hbm,
                    idx_v, ca_v, cb_v, sem):
    c = lax.axis_index("c")
    s = lax.axis_index("s")

    @pl.when(jnp.logical_and(c == 0, s == 0))
    def _():
        pltpu.sync_copy(t_hbm, idx_v)
        pltpu.async_copy(ta_hbm.at[idx_v], ca_v, sem).wait()
        pltpu.async_copy(tb_hbm.at[idx_v], cb_v, sem).wait()
        pltpu.sync_copy(ca_v, ca_hbm)
        pltpu.sync_copy(cb_v, cb_hbm)


def _sc_gather(t, ta, tb):
    B = t.shape[0]
    mesh = plsc.VectorSubcoreMesh(core_axis_name="c", subcore_axis_name="s")
    return pl.kernel(
        _sc_gather_body,
        mesh=mesh,
        out_type=(jax.ShapeDtypeStruct((B,), jnp.float32),
                  jax.ShapeDtypeStruct((B,), jnp.float32)),
        scratch_types=[
            pltpu.VMEM((B,), jnp.int32),
            pltpu.VMEM((B,), jnp.float32),
            pltpu.VMEM((B,), jnp.float32),
            pltpu.SemaphoreType.DMA,
        ],
    )(t, ta, tb)


def _tc_body(ca_ref, cb_ref, x_ref, n_ref, o_ref):
    b = pl.program_id(0)
    o_ref[...] = ca_ref[b] * x_ref[...] + cb_ref[b] * n_ref[...]


@jax.jit
def kernel(x_start, t, noise):
    B, C, H, W = x_start.shape
    ta, tb = _tables()
    ca, cb = _sc_gather(t, ta, tb)

    grid_spec = pltpu.PrefetchScalarGridSpec(
        num_scalar_prefetch=2,
        grid=(B,),
        in_specs=[
            pl.BlockSpec((1, C, H, W), lambda b, *_: (b, 0, 0, 0)),
            pl.BlockSpec((1, C, H, W), lambda b, *_: (b, 0, 0, 0)),
        ],
        out_specs=pl.BlockSpec((1, C, H, W), lambda b, *_: (b, 0, 0, 0)),
    )
    return pl.pallas_call(
        _tc_body,
        grid_spec=grid_spec,
        out_shape=jax.ShapeDtypeStruct((B, C, H, W), jnp.float32),
    )(ca, cb, x_start, noise)


# speedup vs baseline: 1.0015x; 1.0015x over previous
import functools

import jax
import jax.numpy as jnp
from jax import lax
from jax.experimental import pallas as pl
from jax.experimental.pallas import tpu as pltpu
from jax.experimental.pallas import tpu_sc as plsc

_TIMESTEPS = 100


def _tables():
    scale = 1000.0 / _TIMESTEPS
    betas = jnp.linspace(scale * 0.0001, scale * 0.02, _TIMESTEPS)
    alphas_cumprod = jnp.cumprod(1.0 - betas)
    return (jnp.sqrt(alphas_cumprod).astype(jnp.float32),
            jnp.sqrt(1.0 - alphas_cumprod).astype(jnp.float32))


def _sc_gather_body(t_hbm, ta_hbm, tb_hbm, ca_hbm, cb_hbm,
                    idx_v, ca_v, cb_v, sem):
    c = lax.axis_index("c")
    s = lax.axis_index("s")

    @pl.when(jnp.logical_and(c == 0, s == 0))
    def _():
        pltpu.sync_copy(t_hbm, idx_v)
        pltpu.async_copy(ta_hbm.at[idx_v], ca_v, sem).wait()
        pltpu.async_copy(tb_hbm.at[idx_v], cb_v, sem).wait()
        pltpu.sync_copy(ca_v, ca_hbm)
        pltpu.sync_copy(cb_v, cb_hbm)


def _sc_gather(t, ta, tb):
    B = t.shape[0]
    mesh = plsc.VectorSubcoreMesh(core_axis_name="c", subcore_axis_name="s")
    return pl.kernel(
        _sc_gather_body,
        mesh=mesh,
        out_type=(jax.ShapeDtypeStruct((B,), jnp.float32),
                  jax.ShapeDtypeStruct((B,), jnp.float32)),
        scratch_types=[
            pltpu.VMEM((B,), jnp.int32),
            pltpu.VMEM((B,), jnp.float32),
            pltpu.VMEM((B,), jnp.float32),
            pltpu.SemaphoreType.DMA,
        ],
    )(t, ta, tb)


def _tc_body(rows_per_block, ca_ref, cb_ref, x_ref, n_ref, o_ref):
    g = pl.program_id(0)
    for i in range(rows_per_block):
        b = g * rows_per_block + i
        o_ref[i] = ca_ref[b] * x_ref[i] + cb_ref[b] * n_ref[i]


@jax.jit
def kernel(x_start, t, noise):
    B, C, H, W = x_start.shape
    ta, tb = _tables()
    ca, cb = _sc_gather(t, ta, tb)

    rows = 2
    grid_spec = pltpu.PrefetchScalarGridSpec(
        num_scalar_prefetch=2,
        grid=(B // rows,),
        in_specs=[
            pl.BlockSpec((rows, C, H, W), lambda g, *_: (g, 0, 0, 0)),
            pl.BlockSpec((rows, C, H, W), lambda g, *_: (g, 0, 0, 0)),
        ],
        out_specs=pl.BlockSpec((rows, C, H, W), lambda g, *_: (g, 0, 0, 0)),
    )
    return pl.pallas_call(
        functools.partial(_tc_body, rows),
        grid_spec=grid_spec,
        out_shape=jax.ShapeDtypeStruct((B, C, H, W), jnp.float32),
    )(ca, cb, x_start, noise)


# speedup vs baseline: 1.0033x; 1.0018x over previous
import functools

import jax
import jax.numpy as jnp
from jax import lax
from jax.experimental import pallas as pl
from jax.experimental.pallas import tpu as pltpu
from jax.experimental.pallas import tpu_sc as plsc

_TIMESTEPS = 100


def _tables():
    scale = 1000.0 / _TIMESTEPS
    betas = jnp.linspace(scale * 0.0001, scale * 0.02, _TIMESTEPS)
    alphas_cumprod = jnp.cumprod(1.0 - betas)
    return (jnp.sqrt(alphas_cumprod).astype(jnp.float32),
            jnp.sqrt(1.0 - alphas_cumprod).astype(jnp.float32))


def _sc_gather_body(B, t_hbm, tab_hbm, coef_hbm, t_v, idx_v, coef_v, sem):
    c = lax.axis_index("c")
    s = lax.axis_index("s")

    @pl.when(jnp.logical_and(c == 0, s == 0))
    def _():
        pltpu.sync_copy(t_hbm, t_v)
        for j in range(B // 16):
            v = t_v[pl.ds(j * 16, 16)]
            idx_v[pl.ds(j * 16, 16)] = v
            idx_v[pl.ds(B + j * 16, 16)] = v + 128
        pltpu.async_copy(tab_hbm.at[idx_v], coef_v, sem).wait()
        pltpu.sync_copy(coef_v, coef_hbm)


def _sc_gather(t, tab):
    B = t.shape[0]
    mesh = plsc.VectorSubcoreMesh(core_axis_name="c", subcore_axis_name="s")
    return pl.kernel(
        functools.partial(_sc_gather_body, B),
        mesh=mesh,
        out_type=jax.ShapeDtypeStruct((2 * B,), jnp.float32),
        scratch_types=[
            pltpu.VMEM((B,), jnp.int32),
            pltpu.VMEM((2 * B,), jnp.int32),
            pltpu.VMEM((2 * B,), jnp.float32),
            pltpu.SemaphoreType.DMA,
        ],
    )(t, tab)


def _tc_body(rows_per_block, batch, coef_ref, x_ref, n_ref, o_ref):
    g = pl.program_id(0)
    for i in range(rows_per_block):
        b = g * rows_per_block + i
        o_ref[i] = coef_ref[b] * x_ref[i] + coef_ref[batch + b] * n_ref[i]


@jax.jit
def kernel(x_start, t, noise):
    B, C, H, W = x_start.shape
    ta, tb = _tables()
    tab = jnp.zeros((256,), jnp.float32).at[0:100].set(ta).at[128:228].set(tb)
    coef = _sc_gather(t, tab)

    rows = 2
    grid_spec = pltpu.PrefetchScalarGridSpec(
        num_scalar_prefetch=1,
        grid=(B // rows,),
        in_specs=[
            pl.BlockSpec((rows, C, H, W), lambda g, *_: (g, 0, 0, 0)),
            pl.BlockSpec((rows, C, H, W), lambda g, *_: (g, 0, 0, 0)),
        ],
        out_specs=pl.BlockSpec((rows, C, H, W), lambda g, *_: (g, 0, 0, 0)),
    )
    return pl.pallas_call(
        functools.partial(_tc_body, rows, B),
        grid_spec=grid_spec,
        out_shape=jax.ShapeDtypeStruct((B, C, H, W), jnp.float32),
    )(coef, x_start, noise)


# speedup vs baseline: 1.0105x; 1.0071x over previous
import functools

import jax
import jax.numpy as jnp
from jax import lax
from jax.experimental import pallas as pl
from jax.experimental.pallas import tpu as pltpu
from jax.experimental.pallas import tpu_sc as plsc

_TIMESTEPS = 100
_HEAD = 8
_ROWS = 2


def _tables():
    scale = 1000.0 / _TIMESTEPS
    betas = jnp.linspace(scale * 0.0001, scale * 0.02, _TIMESTEPS)
    alphas_cumprod = jnp.cumprod(1.0 - betas)
    return (jnp.sqrt(alphas_cumprod).astype(jnp.float32),
            jnp.sqrt(1.0 - alphas_cumprod).astype(jnp.float32))


def _sc_gather_body(B, t_hbm, tab_hbm, coef_hbm, t_v, idx_v, coef_v, sem):
    c = lax.axis_index("c")
    s = lax.axis_index("s")

    @pl.when(jnp.logical_and(c == 0, s == 0))
    def _():
        pltpu.sync_copy(t_hbm, t_v)
        for j in range(B // 16):
            v = t_v[pl.ds(j * 16, 16)]
            idx_v[pl.ds(j * 16, 16)] = v
            idx_v[pl.ds(B + j * 16, 16)] = v + 128
        pltpu.async_copy(tab_hbm.at[idx_v], coef_v, sem).wait()
        pltpu.sync_copy(coef_v, coef_hbm)


def _sc_gather(t, tab):
    B = t.shape[0]
    mesh = plsc.VectorSubcoreMesh(core_axis_name="c", subcore_axis_name="s")
    return pl.kernel(
        functools.partial(_sc_gather_body, B),
        mesh=mesh,
        out_type=jax.ShapeDtypeStruct((2 * B,), jnp.float32),
        scratch_types=[
            pltpu.VMEM((B,), jnp.int32),
            pltpu.VMEM((2 * B,), jnp.int32),
            pltpu.VMEM((2 * B,), jnp.float32),
            pltpu.SemaphoreType.DMA,
        ],
    )(t, tab)


def _tc_head_body(t_ref, ta_ref, tb_ref, x_ref, n_ref, o_ref):
    g = pl.program_id(0)
    for i in range(_ROWS):
        b = g * _ROWS + i
        tt = t_ref[b]
        o_ref[i] = ta_ref[tt] * x_ref[i] + tb_ref[tt] * n_ref[i]


def _tc_tail_body(batch, coef_ref, prev_ref, x_ref, n_ref, o_ref):
    del prev_ref
    g = pl.program_id(0)
    for i in range(_ROWS):
        b = _HEAD + g * _ROWS + i
        o_ref[i] = coef_ref[b] * x_ref[i] + coef_ref[batch + b] * n_ref[i]


@jax.jit
def kernel(x_start, t, noise):
    B, C, H, W = x_start.shape
    ta, tb = _tables()
    tab = jnp.zeros((256,), jnp.float32).at[0:100].set(ta).at[128:228].set(tb)

    coef = _sc_gather(t, tab)

    blk = pl.BlockSpec((_ROWS, C, H, W), lambda g, *_: (g, 0, 0, 0))
    head = pl.pallas_call(
        _tc_head_body,
        grid_spec=pltpu.PrefetchScalarGridSpec(
            num_scalar_prefetch=3,
            grid=(_HEAD // _ROWS,),
            in_specs=[blk, blk],
            out_specs=blk,
        ),
        out_shape=jax.ShapeDtypeStruct((B, C, H, W), jnp.float32),
    )(t, ta, tb, x_start, noise)

    off = _HEAD // _ROWS
    blk_t = pl.BlockSpec((_ROWS, C, H, W), lambda g, *_: (g + off, 0, 0, 0))
    return pl.pallas_call(
        functools.partial(_tc_tail_body, B),
        grid_spec=pltpu.PrefetchScalarGridSpec(
            num_scalar_prefetch=1,
            grid=((B - _HEAD) // _ROWS,),
            in_specs=[
                pl.BlockSpec(memory_space=pl.ANY),
                blk_t,
                blk_t,
            ],
            out_specs=blk_t,
        ),
        out_shape=jax.ShapeDtypeStruct((B, C, H, W), jnp.float32),
        input_output_aliases={1: 0},
    )(coef, head, x_start, noise)


# speedup vs baseline: 1.0180x; 1.0074x over previous
import functools

import jax
import jax.numpy as jnp
from jax import lax
from jax.experimental import pallas as pl
from jax.experimental.pallas import tpu as pltpu
from jax.experimental.pallas import tpu_sc as plsc

_TIMESTEPS = 100
_HEAD = 8
_ROWS = 2


def _tables():
    scale = 1000.0 / _TIMESTEPS
    betas = jnp.linspace(scale * 0.0001, scale * 0.02, _TIMESTEPS)
    alphas_cumprod = jnp.cumprod(1.0 - betas)
    return (jnp.sqrt(alphas_cumprod).astype(jnp.float32),
            jnp.sqrt(1.0 - alphas_cumprod).astype(jnp.float32))


def _sc_gather_body(B, t_hbm, tab_hbm, coef_hbm, t_v, idx_v, coef_v, sem):
    c = lax.axis_index("c")
    s = lax.axis_index("s")

    @pl.when(jnp.logical_and(c == 0, s == 0))
    def _():
        pltpu.sync_copy(t_hbm, t_v)
        for j in range(B // 16):
            v = t_v[pl.ds(j * 16, 16)]
            idx_v[pl.ds(j * 16, 16)] = v
            idx_v[pl.ds(B + j * 16, 16)] = v + 128
        pltpu.async_copy(tab_hbm.at[idx_v], coef_v, sem).wait()
        pltpu.sync_copy(coef_v, coef_hbm)


def _sc_gather(t, tab):
    B = t.shape[0]
    mesh = plsc.VectorSubcoreMesh(core_axis_name="c", subcore_axis_name="s",
                                  num_cores=1)
    return pl.kernel(
        functools.partial(_sc_gather_body, B),
        mesh=mesh,
        out_type=jax.ShapeDtypeStruct((2 * B,), jnp.float32),
        scratch_types=[
            pltpu.VMEM((B,), jnp.int32),
            pltpu.VMEM((2 * B,), jnp.int32),
            pltpu.VMEM((2 * B,), jnp.float32),
            pltpu.SemaphoreType.DMA,
        ],
    )(t, tab)


def _tc_head_body(t_ref, ta_ref, tb_ref, x_ref, n_ref, o_ref):
    g = pl.program_id(0)
    for i in range(_ROWS):
        b = g * _ROWS + i
        tt = t_ref[b]
        o_ref[i] = ta_ref[tt] * x_ref[i] + tb_ref[tt] * n_ref[i]


def _tc_tail_body(batch, coef_ref, prev_ref, x_ref, n_ref, o_ref):
    del prev_ref
    g = pl.program_id(0)
    for i in range(_ROWS):
        b = _HEAD + g * _ROWS + i
        o_ref[i] = coef_ref[b] * x_ref[i] + coef_ref[batch + b] * n_ref[i]


@jax.jit
def kernel(x_start, t, noise):
    B, C, H, W = x_start.shape
    ta, tb = _tables()
    tab = jnp.zeros((256,), jnp.float32).at[0:100].set(ta).at[128:228].set(tb)

    coef = _sc_gather(t, tab)

    blk = pl.BlockSpec((_ROWS, C, H, W), lambda g, *_: (g, 0, 0, 0))
    head = pl.pallas_call(
        _tc_head_body,
        grid_spec=pltpu.PrefetchScalarGridSpec(
            num_scalar_prefetch=3,
            grid=(_HEAD // _ROWS,),
            in_specs=[blk, blk],
            out_specs=blk,
        ),
        out_shape=jax.ShapeDtypeStruct((B, C, H, W), jnp.float32),
    )(t, ta, tb, x_start, noise)

    off = _HEAD // _ROWS
    blk_t = pl.BlockSpec((_ROWS, C, H, W), lambda g, *_: (g + off, 0, 0, 0))
    return pl.pallas_call(
        functools.partial(_tc_tail_body, B),
        grid_spec=pltpu.PrefetchScalarGridSpec(
            num_scalar_prefetch=1,
            grid=((B - _HEAD) // _ROWS,),
            in_specs=[
                pl.BlockSpec(memory_space=pl.ANY),
                blk_t,
                blk_t,
            ],
            out_specs=blk_t,
        ),
        out_shape=jax.ShapeDtypeStruct((B, C, H, W), jnp.float32),
        input_output_aliases={1: 0},
    )(coef, head, x_start, noise)


# speedup vs baseline: 1.0903x; 1.0710x over previous
import functools

import jax
import jax.numpy as jnp
from jax import lax
from jax.experimental import pallas as pl
from jax.experimental.pallas import tpu as pltpu
from jax.experimental.pallas import tpu_sc as plsc

_TIMESTEPS = 100
_HEAD = 8
_ROWS = 2


def _tables():
    scale = 1000.0 / _TIMESTEPS
    betas = jnp.linspace(scale * 0.0001, scale * 0.02, _TIMESTEPS)
    alphas_cumprod = jnp.cumprod(1.0 - betas)
    return (jnp.sqrt(alphas_cumprod).astype(jnp.float32),
            jnp.sqrt(1.0 - alphas_cumprod).astype(jnp.float32))


def _sc_gather_body(B, t_hbm, tab_hbm, coef_hbm, t_v, idx_v, coef_v, sem):
    c = lax.axis_index("c")
    s = lax.axis_index("s")

    @pl.when(jnp.logical_and(c == 0, s == 0))
    def _():
        pltpu.sync_copy(t_hbm, t_v)
        for j in range(B // 16):
            v = t_v[pl.ds(j * 16, 16)]
            idx_v[pl.ds(j * 16, 16)] = v
            idx_v[pl.ds(B + j * 16, 16)] = v + 128
        pltpu.async_copy(tab_hbm.at[idx_v], coef_v, sem).wait()
        pltpu.sync_copy(coef_v, coef_hbm)


def _sc_gather(t, tab):
    B = t.shape[0]
    mesh = plsc.VectorSubcoreMesh(core_axis_name="c", subcore_axis_name="s",
                                  num_cores=1)
    return pl.kernel(
        functools.partial(_sc_gather_body, B),
        mesh=mesh,
        out_type=jax.ShapeDtypeStruct((2 * B,), jnp.float32),
        scratch_types=[
            pltpu.VMEM((B,), jnp.int32),
            pltpu.VMEM((2 * B,), jnp.int32),
            pltpu.VMEM((2 * B,), jnp.float32),
            pltpu.SemaphoreType.DMA,
        ],
    )(t, tab)


def _tc_head_body(t_ref, ta_ref, tb_ref, x_ref, n_ref, o_ref):
    g = pl.program_id(0)
    for i in range(_ROWS):
        b = g * _ROWS + i
        tt = t_ref[b]
        o_ref[i] = ta_ref[tt] * x_ref[i] + tb_ref[tt] * n_ref[i]


def _tc_tail_body(batch, t_ref, ta_ref, tb_ref, prev_ref, x_ref, n_ref, o_ref):
    del prev_ref, batch
    g = pl.program_id(0)
    for i in range(_ROWS):
        b = _HEAD + g * _ROWS + i
        tt = t_ref[b]
        o_ref[i] = ta_ref[tt] * x_ref[i] + tb_ref[tt] * n_ref[i]


@jax.jit
def kernel(x_start, t, noise):
    B, C, H, W = x_start.shape
    ta, tb = _tables()
    tab = jnp.zeros((256,), jnp.float32).at[0:100].set(ta).at[128:228].set(tb)

    del tab

    blk = pl.BlockSpec((_ROWS, C, H, W), lambda g, *_: (g, 0, 0, 0))
    head = pl.pallas_call(
        _tc_head_body,
        grid_spec=pltpu.PrefetchScalarGridSpec(
            num_scalar_prefetch=3,
            grid=(_HEAD // _ROWS,),
            in_specs=[blk, blk],
            out_specs=blk,
        ),
        out_shape=jax.ShapeDtypeStruct((B, C, H, W), jnp.float32),
    )(t, ta, tb, x_start, noise)

    off = _HEAD // _ROWS
    blk_t = pl.BlockSpec((_ROWS, C, H, W), lambda g, *_: (g + off, 0, 0, 0))
    return pl.pallas_call(
        functools.partial(_tc_tail_body, B),
        grid_spec=pltpu.PrefetchScalarGridSpec(
            num_scalar_prefetch=3,
            grid=((B - _HEAD) // _ROWS,),
            in_specs=[
                pl.BlockSpec(memory_space=pl.ANY),
                blk_t,
                blk_t,
            ],
            out_specs=blk_t,
        ),
        out_shape=jax.ShapeDtypeStruct((B, C, H, W), jnp.float32),
        input_output_aliases={3: 0},
    )(t, ta, tb, head, x_start, noise)
